# baseline (speedup 1.0000x reference)
import functools

import jax
import jax.numpy as jnp
from jax import lax
from jax.experimental import pallas as pl
from jax.experimental.pallas import tpu as pltpu
from jax.experimental.pallas import tpu_sc as plsc

_EPS = 1e-8
_MARGIN = 0.6
_C = 256
_F = 32
_HW = _F * _F
_IMG = 256
_NW = 32
_CELLS = 64


def _tc_kernel(sq_hbm, rk_hbm, tm_hbm, xq_v, xr_v, tm_v, sem, osem):
    B = 2
    cps = []
    for b in range(B):
        cps.append(pltpu.async_copy(sq_hbm.at[b], xq_v.at[b], sem))
        cps.append(pltpu.async_copy(rk_hbm.at[b], xr_v.at[b], sem))
    ocps = []
    row = jax.lax.broadcasted_iota(jnp.int32, (_HW, _HW), 0)
    for b in range(B):
        cps[2 * b].wait()
        cps[2 * b + 1].wait()
        xq = xq_v[b]
        xr = xr_v[b]
        qn = jnp.maximum(jnp.sqrt(jnp.sum(xq * xq, axis=1, keepdims=True)), _EPS)
        rn = jnp.maximum(jnp.sqrt(jnp.sum(xr * xr, axis=1, keepdims=True)), _EPS)
        xqn = xq / qn
        xrn = xr / rn
        st = jax.lax.dot_general(
            xrn, xqn, (((1,), (1,)), ((), ())),
            preferred_element_type=jnp.float32,
            precision=jax.lax.Precision.DEFAULT,
        )
        m1 = jnp.min(st, axis=0, keepdims=True)
        j1 = jnp.min(jnp.where(st == m1, row, _HW), axis=0, keepdims=True)
        s1 = jnp.where(row == j1, jnp.inf, st)
        m2 = jnp.min(s1, axis=0, keepdims=True)
        j2 = jnp.min(jnp.where(s1 == m2, row, _HW), axis=0, keepdims=True)
        s2 = jnp.where(row == j2, jnp.inf, s1)
        m3 = jnp.min(s2, axis=0, keepdims=True)
        bot3 = m1 + m2 + m3
        tm_v[b] = (bot3 + _MARGIN) - st
        ocps.append(pltpu.async_copy(tm_v.at[b], tm_hbm.at[b], osem))
    for c in ocps:
        c.wait()


def _sc_kernel(tm_hbm, g_hbm, out_hbm, tm_v, g_v, o_v, sem):
    wid = lax.axis_index("s") * 2 + lax.axis_index("c")
    b = wid // 16
    chunk = wid % 16
    g0 = chunk * _CELLS
    fy0 = chunk * 2
    copies = [
        pltpu.async_copy(
            tm_hbm.at[b, g0 + k, :], tm_v.at[pl.ds(k * _HW, _HW)], sem
        )
        for k in range(_CELLS)
    ]
    for r in range(2):
        for cplane in range(2):
            pltpu.sync_copy(
                g_hbm.at[b, 4 + 8 * (fy0 + r), cplane, :],
                g_v.at[pl.ds((2 * r + cplane) * _IMG, _IMG)],
            )
    for c in copies:
        c.wait()
    num = jnp.zeros((16,), jnp.float32)
    den = jnp.zeros((16,), jnp.float32)
    iota = lax.iota(jnp.int32, 16)
    for v in range(4):
        c = iota + (v * 16)
        r = c >> 5
        fx = c & 31
        gidx = r * (2 * _IMG) + (fx << 3) + 4
        gx = plsc.load_gather(g_v, [gidx])
        gy = plsc.load_gather(g_v, [gidx + _IMG])
        px = gx * 256.0
        py = gy * 256.0
        tx = px.astype(jnp.int32)
        ty = py.astype(jnp.int32)
        xmin = tx - (tx.astype(jnp.float32) > px).astype(jnp.int32)
        ymin = ty - (ty.astype(jnp.float32) > py).astype(jnp.int32)
        valid = (xmin >= 0) & (ymin >= 0) & (xmin <= 255) & (ymin <= 255)
        x0 = xmin >> 3
        x1 = (xmin + 1) >> 3
        y0 = ymin >> 3
        y1 = (ymin + 1) >> 3
        mx0 = (x0 >= 0) & (x0 <= _F)
        mx1 = (x1 != x0) & (x1 >= 0) & (x1 <= _F)
        my0 = (y0 >= 0) & (y0 <= _F)
        my1 = (y1 != y0) & (y1 >= 0) & (y1 <= _F)
        for xs, ys, mj in ((x0, y0, mx0 & my0), (x0, y1, mx0 & my1),
                           (x1, y0, mx1 & my0), (x1, y1, mx1 & my1)):
            ia = jnp.clip(ys, 0, _F - 1) * _F + jnp.clip(xs, 0, _F - 1)
            val = plsc.load_gather(tm_v, [c * _HW + ia])
            m = mj & valid
            num = num + jnp.where(m, jnp.maximum(val, 0.0), 0.0)
            den = den + jnp.where(m, 1.0, 0.0)
    o_v[pl.ds(0, 16)] = num
    o_v[pl.ds(16, 16)] = den
    pltpu.sync_copy(o_v, out_hbm.at[wid])


def kernel(sketch_query_vectors, ref_key_vectors, G):
    B = sketch_query_vectors.shape[0]
    sq = jnp.transpose(sketch_query_vectors, (0, 2, 3, 1)).reshape(B, _HW, _C)
    rk = jnp.transpose(ref_key_vectors, (0, 2, 3, 1)).reshape(B, _HW, _C)
    g2 = jnp.transpose(G, (0, 1, 3, 2))
    tm = pl.pallas_call(
        _tc_kernel,
        in_specs=[
            pl.BlockSpec(memory_space=pl.ANY),
            pl.BlockSpec(memory_space=pl.ANY),
        ],
        out_specs=pl.BlockSpec(memory_space=pl.ANY),
        out_shape=jax.ShapeDtypeStruct((B, _HW, _HW), jnp.float32),
        scratch_shapes=[
            pltpu.VMEM((B, _HW, _C), jnp.float32),
            pltpu.VMEM((B, _HW, _C), jnp.float32),
            pltpu.VMEM((B, _HW, _HW), jnp.float32),
            pltpu.SemaphoreType.DMA,
            pltpu.SemaphoreType.DMA,
        ],
    )(sq, rk)
    sc_fn = functools.partial(
        pl.kernel,
        mesh=plsc.VectorSubcoreMesh(core_axis_name="c", subcore_axis_name="s"),
        compiler_params=pltpu.CompilerParams(needs_layout_passes=False),
        out_type=jax.ShapeDtypeStruct((_NW, 32), jnp.float32),
        scratch_types=[
            pltpu.VMEM((_CELLS * _HW,), jnp.float32),
            pltpu.VMEM((4 * _IMG,), jnp.float32),
            pltpu.VMEM((32,), jnp.float32),
            pltpu.SemaphoreType.DMA,
        ],
    )(_sc_kernel)
    parts = sc_fn(tm, g2)
    return parts[:, :16].sum() / (1e-6 + parts[:, 16:].sum())


# speedup vs baseline: 2.5995x; 2.5995x over previous
import jax
import jax.numpy as jnp
from jax.experimental import pallas as pl
from jax.experimental.pallas import tpu as pltpu

_EPS = 1e-8
_MARGIN = 0.6
_C = 256
_F = 32
_HW = _F * _F
_IMG = 256


def _flatten_grid(m):
    colj = jax.lax.broadcasted_iota(jnp.int32, (_F, _HW), 1)
    rowi = jax.lax.broadcasted_iota(jnp.int32, (_F, _HW), 0)
    t = jnp.where((colj & (_F - 1)) == rowi, 1.0, 0.0)
    x = jax.lax.dot_general(
        m, t, (((1,), (0,)), ((), ())),
        preferred_element_type=jnp.float32,
        precision=jax.lax.Precision.HIGHEST,
    )
    return jnp.sum(jnp.where((colj >> 5) == rowi, x, 0.0), axis=0, keepdims=True)


def _triplet_kernel(sq_hbm, rk_hbm, g_hbm, out_ref, xq_v, xr_v, g_v, sem, gsem):
    B = 2
    cps = []
    for b in range(B):
        cps.append(pltpu.async_copy(sq_hbm.at[b], xq_v.at[b], sem))
        cps.append(pltpu.async_copy(rk_hbm.at[b], xr_v.at[b], sem))
    gcps = []
    for b in range(B):
        for fy in range(_F):
            gcps.append(pltpu.async_copy(
                g_hbm.at[b, 4 + 8 * fy], g_v.at[b, fy], gsem))
    num = jnp.float32(0.0)
    den = jnp.float32(0.0)
    row = jax.lax.broadcasted_iota(jnp.int32, (_HW, _HW), 0)
    selc = jax.lax.broadcasted_iota(jnp.int32, (_IMG, _F), 0)
    self_ = jax.lax.broadcasted_iota(jnp.int32, (_IMG, _F), 1)
    sel = jnp.where(selc == 8 * self_ + 4, 1.0, 0.0)
    for c in gcps:
        c.wait()
    for b in range(B):
        cps[2 * b].wait()
        cps[2 * b + 1].wait()
        xq = xq_v[b]
        xr = xr_v[b]
        qn = jnp.maximum(jnp.sqrt(jnp.sum(xq * xq, axis=1, keepdims=True)), _EPS)
        rn = jnp.maximum(jnp.sqrt(jnp.sum(xr * xr, axis=1, keepdims=True)), _EPS)
        xqn = xq / qn
        xrn = xr / rn
        sims = jax.lax.dot_general(
            xqn, xrn, (((1,), (1,)), ((), ())),
            preferred_element_type=jnp.float32,
            precision=jax.lax.Precision.DEFAULT,
        )
        m1 = jnp.min(sims, axis=1, keepdims=True)
        eq1 = sims == m1
        c1 = jnp.sum(jnp.where(eq1, 1.0, 0.0), axis=1, keepdims=True)
        s1 = jnp.where(eq1, jnp.inf, sims)
        m2 = jnp.min(s1, axis=1, keepdims=True)
        eq2 = s1 == m2
        c2 = jnp.sum(jnp.where(eq2, 1.0, 0.0), axis=1, keepdims=True)
        s2 = jnp.where(eq2, jnp.inf, s1)
        m3 = jnp.min(s2, axis=1, keepdims=True)
        n1 = jnp.minimum(c1, 3.0)
        n2 = jnp.minimum(c2, 3.0 - n1)
        n3 = 3.0 - n1 - n2
        m2z = jnp.where(m2 == jnp.inf, 0.0, m2)
        m3z = jnp.where(m3 == jnp.inf, 0.0, m3)
        bot3 = n1 * m1 + n2 * m2z + n3 * m3z

        hinge = jnp.maximum(bot3 - sims + _MARGIN, 0.0)

        gx = _flatten_grid(jax.lax.dot_general(
            g_v[b, :, 0, :], sel, (((1,), (0,)), ((), ())),
            preferred_element_type=jnp.float32,
            precision=jax.lax.Precision.HIGHEST))
        gy = _flatten_grid(jax.lax.dot_general(
            g_v[b, :, 1, :], sel, (((1,), (0,)), ((), ())),
            preferred_element_type=jnp.float32,
            precision=jax.lax.Precision.HIGHEST))

        xmin = jnp.floor(gx * _IMG).astype(jnp.int32)
        ymin = jnp.floor(gy * _IMG).astype(jnp.int32)
        valid = (xmin >= 0) & (ymin >= 0) & (xmin + 1 <= _IMG) & (ymin + 1 <= _IMG)
        x0 = jnp.floor_divide(xmin, 8)
        x1 = jnp.floor_divide(xmin + 1, 8)
        y0 = jnp.floor_divide(ymin, 8)
        y1 = jnp.floor_divide(ymin + 1, 8)
        mx0 = (x0 >= 0) & (x0 <= _F)
        mx1 = (x1 != x0) & (x1 >= 0) & (x1 <= _F)
        my0 = (y0 >= 0) & (y0 <= _F)
        my1 = (y1 != y0) & (y1 >= 0) & (y1 <= _F)

        w = jnp.zeros((_HW, _HW), jnp.float32)
        for xs, ys, mj in ((x0, y0, mx0 & my0), (x0, y1, mx0 & my1),
                           (x1, y0, mx1 & my0), (x1, y1, mx1 & my1)):
            ia = jnp.clip(ys, 0, _F - 1) * _F + jnp.clip(xs, 0, _F - 1)
            mjv = mj & valid
            tj = jnp.where(mjv, ia, -1)
            w = w + jnp.where(row == tj, 1.0, 0.0)
            den = den + jnp.sum(mjv.astype(jnp.float32))
        num = num + jnp.sum(w * hinge)
    out_ref[...] = jnp.broadcast_to(num / (1e-6 + den), (1, 1))


def kernel(sketch_query_vectors, ref_key_vectors, G):
    B = sketch_query_vectors.shape[0]
    sq = jnp.transpose(sketch_query_vectors, (0, 2, 3, 1)).reshape(B, _HW, _C)
    rk = jnp.transpose(ref_key_vectors, (0, 2, 3, 1)).reshape(B, _HW, _C)
    g2 = jnp.transpose(G, (0, 1, 3, 2))
    out = pl.pallas_call(
        _triplet_kernel,
        in_specs=[
            pl.BlockSpec(memory_space=pl.ANY),
            pl.BlockSpec(memory_space=pl.ANY),
            pl.BlockSpec(memory_space=pl.ANY),
        ],
        out_specs=pl.BlockSpec(memory_space=pltpu.VMEM),
        out_shape=jax.ShapeDtypeStruct((1, 1), jnp.float32),
        scratch_shapes=[
            pltpu.VMEM((B, _HW, _C), jnp.float32),
            pltpu.VMEM((B, _HW, _C), jnp.float32),
            pltpu.VMEM((B, _F, 2, _IMG), jnp.float32),
            pltpu.SemaphoreType.DMA,
            pltpu.SemaphoreType.DMA,
        ],
    )(sq, rk, g2)
    return out[0, 0]


# speedup vs baseline: 2.6654x; 1.0254x over previous
import jax
import jax.numpy as jnp
from jax.experimental import pallas as pl
from jax.experimental.pallas import tpu as pltpu

_EPS = 1e-8
_MARGIN = 0.6
_C = 256
_F = 32
_HW = _F * _F
_IMG = 256


def _flatten_grid(m):
    colj = jax.lax.broadcasted_iota(jnp.int32, (_F, _HW), 1)
    rowi = jax.lax.broadcasted_iota(jnp.int32, (_F, _HW), 0)
    t = jnp.where((colj & (_F - 1)) == rowi, 1.0, 0.0)
    x = jax.lax.dot_general(
        m, t, (((1,), (0,)), ((), ())),
        preferred_element_type=jnp.float32,
        precision=jax.lax.Precision.HIGHEST,
    )
    return jnp.sum(jnp.where((colj >> 5) == rowi, x, 0.0), axis=0, keepdims=True)


def _triplet_kernel(sq_hbm, rk_hbm, g_hbm, out_ref, xq_v, xr_v, g_v, sem, gsem):
    B = 2
    cps = []
    for b in range(B):
        cps.append(pltpu.async_copy(sq_hbm.at[b], xq_v.at[b], sem))
        cps.append(pltpu.async_copy(rk_hbm.at[b], xr_v.at[b], sem))
    gcps = []
    for b in range(B):
        for fy in range(_F):
            gcps.append(pltpu.async_copy(
                g_hbm.at[b, 4 + 8 * fy], g_v.at[b, fy], gsem))
    num = jnp.float32(0.0)
    den = jnp.float32(0.0)
    row = jax.lax.broadcasted_iota(jnp.int32, (_HW, _HW), 0)
    selc = jax.lax.broadcasted_iota(jnp.int32, (_IMG, _F), 0)
    self_ = jax.lax.broadcasted_iota(jnp.int32, (_IMG, _F), 1)
    sel = jnp.where(selc == 8 * self_ + 4, 1.0, 0.0)
    for c in gcps:
        c.wait()
    for b in range(B):
        cps[2 * b].wait()
        cps[2 * b + 1].wait()
        xq = xq_v[b]
        xr = xr_v[b]
        qn = jnp.maximum(jnp.sqrt(jnp.sum(xq * xq, axis=1, keepdims=True)), _EPS)
        rn = jnp.maximum(jnp.sqrt(jnp.sum(xr * xr, axis=1, keepdims=True)), _EPS)
        xqn = xq / qn
        xrn = xr / rn
        sims = jax.lax.dot_general(
            xqn, xrn, (((1,), (1,)), ((), ())),
            preferred_element_type=jnp.float32,
            precision=jax.lax.Precision.DEFAULT,
        )
        m1 = jnp.min(sims, axis=1, keepdims=True)
        eq1 = sims == m1
        c1 = jnp.sum(jnp.where(eq1, 1.0, 0.0), axis=1, keepdims=True)
        s1 = jnp.where(eq1, jnp.inf, sims)
        m2 = jnp.min(s1, axis=1, keepdims=True)
        eq2 = s1 == m2
        c2 = jnp.sum(jnp.where(eq2, 1.0, 0.0), axis=1, keepdims=True)
        s2 = jnp.where(eq2, jnp.inf, s1)
        m3 = jnp.min(s2, axis=1, keepdims=True)
        n1 = jnp.minimum(c1, 3.0)
        n2 = jnp.minimum(c2, 3.0 - n1)
        n3 = 3.0 - n1 - n2
        m2z = jnp.where(m2 == jnp.inf, 0.0, m2)
        m3z = jnp.where(m3 == jnp.inf, 0.0, m3)
        bot3 = n1 * m1 + n2 * m2z + n3 * m3z

        hinge = jnp.maximum(bot3 - sims + _MARGIN, 0.0)

        gx = _flatten_grid(jax.lax.dot_general(
            g_v[b, :, 0, :], sel, (((1,), (0,)), ((), ())),
            preferred_element_type=jnp.float32,
            precision=jax.lax.Precision.HIGHEST))
        gy = _flatten_grid(jax.lax.dot_general(
            g_v[b, :, 1, :], sel, (((1,), (0,)), ((), ())),
            preferred_element_type=jnp.float32,
            precision=jax.lax.Precision.HIGHEST))

        xmin = jnp.floor(gx * _IMG).astype(jnp.int32)
        ymin = jnp.floor(gy * _IMG).astype(jnp.int32)
        valid = (xmin >= 0) & (ymin >= 0) & (xmin + 1 <= _IMG) & (ymin + 1 <= _IMG)
        x0 = jnp.floor_divide(xmin, 8)
        x1 = jnp.floor_divide(xmin + 1, 8)
        y0 = jnp.floor_divide(ymin, 8)
        y1 = jnp.floor_divide(ymin + 1, 8)
        mx0 = (x0 >= 0) & (x0 <= _F)
        mx1 = (x1 != x0) & (x1 >= 0) & (x1 <= _F)
        my0 = (y0 >= 0) & (y0 <= _F)
        my1 = (y1 != y0) & (y1 >= 0) & (y1 <= _F)

        acc = jnp.zeros((_HW, _HW), jnp.float32)
        for xs, ys, mj in ((x0, y0, mx0 & my0), (x0, y1, mx0 & my1),
                           (x1, y0, mx1 & my0), (x1, y1, mx1 & my1)):
            ia = jnp.clip(ys, 0, _F - 1) * _F + jnp.clip(xs, 0, _F - 1)
            mjv = mj & valid
            tj = jnp.where(mjv, ia, -1)
            acc = acc + jnp.where(row == tj, hinge, 0.0)
            den = den + jnp.sum(mjv.astype(jnp.float32))
        num = num + jnp.sum(acc)
    out_ref[...] = jnp.broadcast_to(num / (1e-6 + den), (1, 1))


def kernel(sketch_query_vectors, ref_key_vectors, G):
    B = sketch_query_vectors.shape[0]
    sq = jnp.transpose(sketch_query_vectors, (0, 2, 3, 1)).reshape(B, _HW, _C)
    rk = jnp.transpose(ref_key_vectors, (0, 2, 3, 1)).reshape(B, _HW, _C)
    g2 = jnp.transpose(G, (0, 1, 3, 2))
    out = pl.pallas_call(
        _triplet_kernel,
        in_specs=[
            pl.BlockSpec(memory_space=pl.ANY),
            pl.BlockSpec(memory_space=pl.ANY),
            pl.BlockSpec(memory_space=pl.ANY),
        ],
        out_specs=pl.BlockSpec(memory_space=pltpu.VMEM),
        out_shape=jax.ShapeDtypeStruct((1, 1), jnp.float32),
        scratch_shapes=[
            pltpu.VMEM((B, _HW, _C), jnp.float32),
            pltpu.VMEM((B, _HW, _C), jnp.float32),
            pltpu.VMEM((B, _F, 2, _IMG), jnp.float32),
            pltpu.SemaphoreType.DMA,
            pltpu.SemaphoreType.DMA,
        ],
    )(sq, rk, g2)
    return out[0, 0]
